# baseline (speedup 1.0000x reference)
import functools

import jax
import jax.numpy as jnp
from jax import lax
from jax.experimental import pallas as pl
from jax.experimental.pallas import tpu as pltpu
from jax.experimental.pallas import tpu_sc as plsc

NC = 2
NS = 16
NW = NC * NS
LANES = 16

VOCAB = 1000000
BATCH = 4096
SEQ = 200
EMBED_DIM = 64

BPW = BATCH // NW
NV = EMBED_DIM // LANES
INV_SEQ = 1.0 / SEQ

VB = 256
NFULL = VOCAB // VB
TAIL_V = VOCAB - NFULL * VB
BLOCKS_PER_W = -(-NFULL // NW)

CHUNK = 128
NCHUNK_W = BPW * SEQ // CHUNK


def _repack_sc(table_t, r_tail):
    mesh = plsc.VectorSubcoreMesh(core_axis_name="c", subcore_axis_name="s")

    @functools.partial(
        pl.kernel,
        mesh=mesh,
        compiler_params=pltpu.CompilerParams(needs_layout_passes=False),
        out_type=jax.ShapeDtypeStruct((VOCAB // 2, 128), jnp.float32),
        scratch_types=[
            pltpu.VMEM((EMBED_DIM, VB), jnp.float32),
            pltpu.VMEM((VB // 2, 128), jnp.float32),
        ],
    )
    def ka(tt_hbm, rt_hbm, r_hbm, tv, tr):
        wid = lax.axis_index("s") * NC + lax.axis_index("c")

        def transpose_block(ncols, base_col, base_row):
            def jloop(j, _):
                for h in range(2):
                    cvec = jnp.zeros((LANES,), jnp.int32) + (2 * j + h)
                    for q in range(NV):
                        rvec = lax.iota(jnp.int32, LANES) + 16 * q
                        v16 = plsc.load_gather(tv, [rvec, cvec])
                        tr[j, pl.ds(64 * h + 16 * q, 16)] = v16
                return _

            lax.fori_loop(0, ncols // 2, jloop, None)
            pltpu.sync_copy(
                tr.at[pl.ds(0, ncols // 2)], r_hbm.at[pl.ds(base_row, ncols // 2)]
            )

        def blk(t, _):
            b = wid + NW * t

            @pl.when(b < NFULL)
            def _():
                v0 = b * VB
                pltpu.sync_copy(tt_hbm.at[:, pl.ds(v0, VB)], tv)
                transpose_block(VB, v0, b * (VB // 2))

            return _

        lax.fori_loop(0, BLOCKS_PER_W, blk, None)

        @pl.when(wid == 0)
        def _tail():
            pltpu.sync_copy(rt_hbm, tr.at[pl.ds(0, 32)])
            pltpu.sync_copy(
                tr.at[pl.ds(0, 32)], r_hbm.at[pl.ds(NFULL * (VB // 2), 32)]
            )

    return ka(table_t, r_tail)


def _pool_sc(r, idx2, par):
    mesh = plsc.VectorSubcoreMesh(core_axis_name="c", subcore_axis_name="s")

    @functools.partial(
        pl.kernel,
        mesh=mesh,
        compiler_params=pltpu.CompilerParams(needs_layout_passes=False),
        out_type=jax.ShapeDtypeStruct((BATCH // 2, 2 * EMBED_DIM), jnp.float32),
        scratch_types=[
            pltpu.VMEM((NCHUNK_W, CHUNK), jnp.int32),
            pltpu.VMEM((NCHUNK_W, CHUNK), jnp.float32),
            pltpu.VMEM((2, CHUNK, 128), jnp.float32),
            pltpu.VMEM((BPW // 2, 2 * EMBED_DIM), jnp.float32),
            pltpu.SemaphoreType.DMA,
            pltpu.SemaphoreType.DMA,
        ],
    )
    def kb(r_hbm, idx_hbm, par_hbm, out_hbm, idx_v, par_v, rows_v, pooled_v,
           sem0, sem1):
        wid = lax.axis_index("s") * NC + lax.axis_index("c")
        cbase = wid * NCHUNK_W
        pltpu.sync_copy(idx_hbm.at[pl.ds(cbase, NCHUNK_W)], idx_v)
        pltpu.sync_copy(par_hbm.at[pl.ds(cbase, NCHUNK_W)], par_v)

        sems = (sem0, sem1)

        def start(c, slot, sem):
            pltpu.async_copy(r_hbm.at[idx_v.at[c]], rows_v.at[slot], sem)

        def wait(c, slot, sem):
            pltpu.make_async_copy(
                r_hbm.at[idx_v.at[c]], rows_v.at[slot], sem
            ).wait()

        def reduce_chunk(c, slot, accs):
            s_raw = (SEQ - (CHUNK * c) % SEQ) % SEQ
            s = jnp.minimum(s_raw, CHUNK)
            flush = jnp.logical_and(s_raw < CHUNK,
                                    jnp.logical_or(c > 0, s_raw > 0))

            def red_range(accs, lo, hi):
                for k in range(CHUNK // LANES):
                    pv16 = par_v[c, pl.ds(16 * k, 16)]

                    def red(rr, a):
                        lane = jnp.zeros((LANES,), jnp.int32) + (rr - 16 * k)
                        pf = lax.gather(
                            pv16,
                            lane[:, None],
                            lax.GatherDimensionNumbers(
                                offset_dims=(),
                                collapsed_slice_dims=(0,),
                                start_index_map=(0,),
                            ),
                            slice_sizes=(1,),
                            mode=lax.GatherScatterMode.PROMISE_IN_BOUNDS,
                        )
                        out = []
                        for j in range(NV):
                            lo_j = rows_v[slot, rr, pl.ds(16 * j, 16)]
                            hi_j = rows_v[slot, rr, pl.ds(64 + 16 * j, 16)]
                            out.append(a[j] + lo_j + pf * (hi_j - lo_j))
                        return tuple(out)

                    klo = jnp.maximum(lo, 16 * k)
                    khi = jnp.minimum(hi, 16 * (k + 1))
                    accs = lax.fori_loop(klo, khi, red, accs)
                return accs

            accs = red_range(accs, jnp.int32(0), s)

            @pl.when(flush)
            def _():
                rf = (CHUNK * c + s - 1) // SEQ
                off = 64 * (rf & 1)
                for j in range(NV):
                    pooled_v[rf >> 1, pl.ds(off + 16 * j, 16)] = (
                        accs[j] * INV_SEQ
                    )

            accs = tuple(
                jnp.where(flush, jnp.zeros((LANES,), jnp.float32), a)
                for a in accs
            )
            return red_range(accs, s, jnp.int32(CHUNK))

        start(0, 0, sem0)

        def body(cp, accs):
            c0 = 2 * cp
            start(c0 + 1, 1, sem1)
            wait(c0, 0, sem0)
            accs = reduce_chunk(c0, 0, accs)

            @pl.when(c0 + 2 < NCHUNK_W)
            def _():
                start(c0 + 2, 0, sem0)

            wait(c0 + 1, 1, sem1)
            return reduce_chunk(c0 + 1, 1, accs)

        accs = lax.fori_loop(
            0, NCHUNK_W // 2, body,
            tuple(jnp.zeros((LANES,), jnp.float32) for _ in range(NV)),
        )
        for j in range(NV):
            pooled_v[BPW // 2 - 1, pl.ds(64 + 16 * j, 16)] = accs[j] * INV_SEQ
        pltpu.sync_copy(
            pooled_v, out_hbm.at[pl.ds(wid * (BPW // 2), BPW // 2)]
        )

    return kb(r, idx2, par)


def _classifier_tc(pooled, W, b2):
    B, D = pooled.shape
    C = W.shape[0]
    BM = 512

    def mm(x_ref, w_ref, b_ref, o_ref):
        o_ref[...] = (
            lax.dot_general(
                x_ref[...],
                w_ref[...],
                (((1,), (1,)), ((), ())),
                preferred_element_type=jnp.float32,
            )
            + b_ref[...]
        )

    return pl.pallas_call(
        mm,
        grid=(B // BM,),
        in_specs=[
            pl.BlockSpec((BM, D), lambda i: (i, 0)),
            pl.BlockSpec((C, D), lambda i: (0, 0)),
            pl.BlockSpec((1, C), lambda i: (0, 0)),
        ],
        out_specs=pl.BlockSpec((BM, C), lambda i: (i, 0)),
        out_shape=jax.ShapeDtypeStruct((B, C), jnp.float32),
    )(pooled, W, b2)


def kernel(x_data, table, W, b):
    x = x_data.astype(jnp.int32)
    idx2 = (x >> 1).reshape(BATCH * SEQ // CHUNK, CHUNK)
    par = (x & 1).astype(jnp.float32).reshape(BATCH * SEQ // CHUNK, CHUNK)
    table_t = jnp.swapaxes(table, 0, 1)
    r_tail = table[NFULL * VB:].reshape(32, 128)
    r = _repack_sc(table_t, r_tail)
    pooled2 = _pool_sc(r, idx2, par)
    pooled = pooled2.reshape(BATCH, EMBED_DIM)
    return _classifier_tc(pooled, W, b.reshape(1, -1))


# speedup vs baseline: 3.0865x; 3.0865x over previous
import functools

import jax
import jax.numpy as jnp
from jax import lax
from jax.experimental import pallas as pl
from jax.experimental.pallas import tpu as pltpu
from jax.experimental.pallas import tpu_sc as plsc

NC = 2
NS = 16
NW = NC * NS
LANES = 16

VOCAB = 1000000
BATCH = 4096
SEQ = 200
EMBED_DIM = 64

BPW = BATCH // NW
NV = EMBED_DIM // LANES
INV_SEQ = 1.0 / SEQ

BR = 1024
NBLK_IN = -(-VOCAB // BR)
NBLK_OUT = -(-NBLK_IN // 2)
RROWS = NBLK_OUT * BR

CHUNK = 128
NCHUNK_W = BPW * SEQ // CHUNK


def _repack_tc(table_t):
    def rep(lo_ref, hi_ref, o_ref):
        o_ref[...] = jnp.concatenate(
            [
                jnp.swapaxes(lo_ref[...], 0, 1),
                jnp.swapaxes(hi_ref[...], 0, 1),
            ],
            axis=1,
        )

    return pl.pallas_call(
        rep,
        grid=(NBLK_OUT,),
        in_specs=[
            pl.BlockSpec((EMBED_DIM, BR), lambda i: (0, 2 * i)),
            pl.BlockSpec(
                (EMBED_DIM, BR),
                lambda i: (0, jnp.minimum(2 * i + 1, NBLK_IN - 1)),
            ),
        ],
        out_specs=pl.BlockSpec((BR, 128), lambda i: (i, 0)),
        out_shape=jax.ShapeDtypeStruct((RROWS, 128), jnp.float32),
    )(table_t, table_t)


def _pool_sc(r64, gidx):
    mesh = plsc.VectorSubcoreMesh(core_axis_name="c", subcore_axis_name="s")

    @functools.partial(
        pl.kernel,
        mesh=mesh,
        compiler_params=pltpu.CompilerParams(use_tc_tiling_on_sc=False),
        out_type=jax.ShapeDtypeStruct((BATCH, EMBED_DIM), jnp.float32),
        scratch_types=[
            pltpu.VMEM((NCHUNK_W, CHUNK), jnp.int32),
            pltpu.VMEM((2, CHUNK, EMBED_DIM), jnp.float32),
            pltpu.VMEM((BPW, EMBED_DIM), jnp.float32),
            pltpu.SemaphoreType.DMA,
            pltpu.SemaphoreType.DMA,
        ],
    )
    def kb(r_hbm, idx_hbm, out_hbm, idx_v, rows_v, pooled_v, sem0, sem1):
        wid = lax.axis_index("s") * NC + lax.axis_index("c")
        cbase = wid * NCHUNK_W
        pltpu.sync_copy(idx_hbm.at[pl.ds(cbase, NCHUNK_W)], idx_v)

        def start(c, slot, sem):
            pltpu.async_copy(r_hbm.at[idx_v.at[c]], rows_v.at[slot], sem)

        def wait(c, slot, sem):
            pltpu.make_async_copy(
                r_hbm.at[idx_v.at[c]], rows_v.at[slot], sem
            ).wait()

        def reduce_chunk(c, slot, accs):
            s_raw = (SEQ - (CHUNK * c) % SEQ) % SEQ
            s = jnp.minimum(s_raw, CHUNK)
            flush = jnp.logical_and(s_raw < CHUNK,
                                    jnp.logical_or(c > 0, s_raw > 0))

            def red(rr, a):
                return tuple(
                    a[j] + rows_v[slot, rr, pl.ds(16 * j, 16)]
                    for j in range(NV)
                )

            accs = lax.fori_loop(jnp.int32(0), s, red, accs)

            @pl.when(flush)
            def _():
                rf = (CHUNK * c + s - 1) // SEQ
                for j in range(NV):
                    pooled_v[rf, pl.ds(16 * j, 16)] = accs[j] * INV_SEQ

            accs = tuple(
                jnp.where(flush, jnp.zeros((LANES,), jnp.float32), a)
                for a in accs
            )
            return lax.fori_loop(s, jnp.int32(CHUNK), red, accs)

        start(0, 0, sem0)

        def body(cp, accs):
            c0 = 2 * cp
            start(c0 + 1, 1, sem1)
            wait(c0, 0, sem0)
            accs = reduce_chunk(c0, 0, accs)

            @pl.when(c0 + 2 < NCHUNK_W)
            def _():
                start(c0 + 2, 0, sem0)

            wait(c0 + 1, 1, sem1)
            return reduce_chunk(c0 + 1, 1, accs)

        accs = lax.fori_loop(
            0, NCHUNK_W // 2, body,
            tuple(jnp.zeros((LANES,), jnp.float32) for _ in range(NV)),
        )
        for j in range(NV):
            pooled_v[BPW - 1, pl.ds(16 * j, 16)] = accs[j] * INV_SEQ
        pltpu.sync_copy(pooled_v, out_hbm.at[pl.ds(wid * BPW, BPW)])

    return kb(r64, gidx)


def _classifier_tc(pooled, W, b2):
    B, D = pooled.shape
    C = W.shape[0]
    BM = 512

    def mm(x_ref, w_ref, b_ref, o_ref):
        o_ref[...] = (
            lax.dot_general(
                x_ref[...],
                w_ref[...],
                (((1,), (1,)), ((), ())),
                preferred_element_type=jnp.float32,
            )
            + b_ref[...]
        )

    return pl.pallas_call(
        mm,
        grid=(B // BM,),
        in_specs=[
            pl.BlockSpec((BM, D), lambda i: (i, 0)),
            pl.BlockSpec((C, D), lambda i: (0, 0)),
            pl.BlockSpec((1, C), lambda i: (0, 0)),
        ],
        out_specs=pl.BlockSpec((BM, C), lambda i: (i, 0)),
        out_shape=jax.ShapeDtypeStruct((B, C), jnp.float32),
    )(pooled, W, b2)


def kernel(x_data, table, W, b):
    x = x_data.astype(jnp.int32)
    bb = x >> 10
    rr = x & (BR - 1)
    gidx = ((bb >> 1) << 11) + 2 * rr + (bb & 1)
    gidx = gidx.reshape(BATCH * SEQ // CHUNK, CHUNK)
    table_t = jnp.swapaxes(table, 0, 1)
    r = _repack_tc(table_t)
    r64 = r.reshape(2 * RROWS, EMBED_DIM)
    pooled = _pool_sc(r64, gidx)
    return _classifier_tc(pooled, W, b.reshape(1, -1))


# speedup vs baseline: 3.1111x; 1.0080x over previous
import functools

import jax
import jax.numpy as jnp
from jax import lax
from jax.experimental import pallas as pl
from jax.experimental.pallas import tpu as pltpu
from jax.experimental.pallas import tpu_sc as plsc

NC = 2
NS = 16
NW = NC * NS
LANES = 16

VOCAB = 1000000
BATCH = 4096
SEQ = 200
EMBED_DIM = 64

BPW = BATCH // NW
NV = EMBED_DIM // LANES
INV_SEQ = 1.0 / SEQ

BR = 1024
NBLK_IN = -(-VOCAB // BR)
NBLK_OUT = -(-NBLK_IN // 2)
RROWS = NBLK_OUT * BR

CHUNK = 128
NCHUNK_W = BPW * SEQ // CHUNK


def _repack_tc(table_t):
    e_lo = jnp.eye(EMBED_DIM, 2 * EMBED_DIM, dtype=jnp.float32)
    e_hi = jnp.eye(EMBED_DIM, 2 * EMBED_DIM, k=EMBED_DIM, dtype=jnp.float32)

    def rep(lo_ref, hi_ref, elo_ref, ehi_ref, o_ref):
        cdims = (((0,), (0,)), ((), ()))
        o_ref[...] = lax.dot_general(
            lo_ref[...], elo_ref[...], cdims,
            preferred_element_type=jnp.float32,
        ) + lax.dot_general(
            hi_ref[...], ehi_ref[...], cdims,
            preferred_element_type=jnp.float32,
        )

    return pl.pallas_call(
        rep,
        grid=(NBLK_OUT,),
        compiler_params=pltpu.CompilerParams(
            fuse_transposed_lhs_in_matmul=True
        ),
        in_specs=[
            pl.BlockSpec((EMBED_DIM, BR), lambda i: (0, 2 * i)),
            pl.BlockSpec(
                (EMBED_DIM, BR),
                lambda i: (0, jnp.minimum(2 * i + 1, NBLK_IN - 1)),
            ),
            pl.BlockSpec((EMBED_DIM, 2 * EMBED_DIM), lambda i: (0, 0)),
            pl.BlockSpec((EMBED_DIM, 2 * EMBED_DIM), lambda i: (0, 0)),
        ],
        out_specs=pl.BlockSpec((BR, 128), lambda i: (i, 0)),
        out_shape=jax.ShapeDtypeStruct((RROWS, 128), jnp.float32),
    )(table_t, table_t, e_lo, e_hi)


def _pool_sc(r64, gidx):
    mesh = plsc.VectorSubcoreMesh(core_axis_name="c", subcore_axis_name="s")

    @functools.partial(
        pl.kernel,
        mesh=mesh,
        compiler_params=pltpu.CompilerParams(use_tc_tiling_on_sc=False),
        out_type=jax.ShapeDtypeStruct((BATCH, EMBED_DIM), jnp.float32),
        scratch_types=[
            pltpu.VMEM((NCHUNK_W, CHUNK), jnp.int32),
            pltpu.VMEM((2, CHUNK, EMBED_DIM), jnp.float32),
            pltpu.VMEM((BPW, EMBED_DIM), jnp.float32),
            pltpu.SemaphoreType.DMA,
            pltpu.SemaphoreType.DMA,
        ],
    )
    def kb(r_hbm, idx_hbm, out_hbm, idx_v, rows_v, pooled_v, sem0, sem1):
        wid = lax.axis_index("s") * NC + lax.axis_index("c")
        cbase = wid * NCHUNK_W
        pltpu.sync_copy(idx_hbm.at[pl.ds(cbase, NCHUNK_W)], idx_v)

        def start(c, slot, sem):
            pltpu.async_copy(r_hbm.at[idx_v.at[c]], rows_v.at[slot], sem)

        def wait(c, slot, sem):
            pltpu.make_async_copy(
                r_hbm.at[idx_v.at[c]], rows_v.at[slot], sem
            ).wait()

        def reduce_chunk(c, slot, accs):
            s_raw = (SEQ - (CHUNK * c) % SEQ) % SEQ
            s = jnp.minimum(s_raw, CHUNK)
            flush = jnp.logical_and(s_raw < CHUNK,
                                    jnp.logical_or(c > 0, s_raw > 0))

            def red(rr, a):
                return tuple(
                    a[j] + rows_v[slot, rr, pl.ds(16 * j, 16)]
                    for j in range(NV)
                )

            accs = lax.fori_loop(jnp.int32(0), s, red, accs)

            @pl.when(flush)
            def _():
                rf = (CHUNK * c + s - 1) // SEQ
                for j in range(NV):
                    pooled_v[rf, pl.ds(16 * j, 16)] = accs[j] * INV_SEQ

            accs = tuple(
                jnp.where(flush, jnp.zeros((LANES,), jnp.float32), a)
                for a in accs
            )
            return lax.fori_loop(s, jnp.int32(CHUNK), red, accs)

        start(0, 0, sem0)

        def body(cp, accs):
            c0 = 2 * cp
            start(c0 + 1, 1, sem1)
            wait(c0, 0, sem0)
            accs = reduce_chunk(c0, 0, accs)

            @pl.when(c0 + 2 < NCHUNK_W)
            def _():
                start(c0 + 2, 0, sem0)

            wait(c0 + 1, 1, sem1)
            return reduce_chunk(c0 + 1, 1, accs)

        accs = lax.fori_loop(
            0, NCHUNK_W // 2, body,
            tuple(jnp.zeros((LANES,), jnp.float32) for _ in range(NV)),
        )
        for j in range(NV):
            pooled_v[BPW - 1, pl.ds(16 * j, 16)] = accs[j] * INV_SEQ
        pltpu.sync_copy(pooled_v, out_hbm.at[pl.ds(wid * BPW, BPW)])

    return kb(r64, gidx)


def _classifier_tc(pooled, W, b2):
    B, D = pooled.shape
    C = W.shape[0]
    BM = 512

    def mm(x_ref, w_ref, b_ref, o_ref):
        o_ref[...] = (
            lax.dot_general(
                x_ref[...],
                w_ref[...],
                (((1,), (1,)), ((), ())),
                preferred_element_type=jnp.float32,
            )
            + b_ref[...]
        )

    return pl.pallas_call(
        mm,
        grid=(B // BM,),
        in_specs=[
            pl.BlockSpec((BM, D), lambda i: (i, 0)),
            pl.BlockSpec((C, D), lambda i: (0, 0)),
            pl.BlockSpec((1, C), lambda i: (0, 0)),
        ],
        out_specs=pl.BlockSpec((BM, C), lambda i: (i, 0)),
        out_shape=jax.ShapeDtypeStruct((B, C), jnp.float32),
    )(pooled, W, b2)


def kernel(x_data, table, W, b):
    x = x_data.astype(jnp.int32)
    bb = x >> 10
    rr = x & (BR - 1)
    gidx = ((bb >> 1) << 11) + 2 * rr + (bb & 1)
    gidx = gidx.reshape(BATCH * SEQ // CHUNK, CHUNK)
    table_t = jnp.swapaxes(table, 0, 1)
    r = _repack_tc(table_t)
    r64 = r.reshape(2 * RROWS, EMBED_DIM)
    pooled = _pool_sc(r64, gidx)
    return _classifier_tc(pooled, W, b.reshape(1, -1))


# speedup vs baseline: 3.7916x; 1.2187x over previous
import functools

import jax
import jax.numpy as jnp
from jax import lax
from jax.experimental import pallas as pl
from jax.experimental.pallas import tpu as pltpu
from jax.experimental.pallas import tpu_sc as plsc

NC = 2
NS = 16
NW = NC * NS
LANES = 16

VOCAB = 1000000
BATCH = 4096
SEQ = 200
EMBED_DIM = 64

BPW = BATCH // NW
NV = EMBED_DIM // LANES
INV_SEQ = 1.0 / SEQ

BR = 2048
NBLK_IN = -(-VOCAB // BR)
NBLK_OUT = -(-NBLK_IN // 2)
RROWS = NBLK_OUT * BR

CHUNK = 128
NCHUNK_W = BPW * SEQ // CHUNK


def _repack_tc(table_t):
    def rep(lo_ref, hi_ref, o_ref):
        o_ref[...] = jnp.concatenate(
            [
                jnp.swapaxes(lo_ref[...], 0, 1),
                jnp.swapaxes(hi_ref[...], 0, 1),
            ],
            axis=1,
        )

    return pl.pallas_call(
        rep,
        grid=(NBLK_OUT,),
        in_specs=[
            pl.BlockSpec((EMBED_DIM, BR), lambda i: (0, 2 * i)),
            pl.BlockSpec(
                (EMBED_DIM, BR),
                lambda i: (0, jnp.minimum(2 * i + 1, NBLK_IN - 1)),
            ),
        ],
        out_specs=pl.BlockSpec((BR, 128), lambda i: (i, 0)),
        out_shape=jax.ShapeDtypeStruct((RROWS, 128), jnp.float32),
    )(table_t, table_t)


def _pool_sc(r64, gidx):
    mesh = plsc.VectorSubcoreMesh(core_axis_name="c", subcore_axis_name="s")

    @functools.partial(
        pl.kernel,
        mesh=mesh,
        compiler_params=pltpu.CompilerParams(use_tc_tiling_on_sc=False),
        out_type=jax.ShapeDtypeStruct((BATCH, EMBED_DIM), jnp.float32),
        scratch_types=[
            pltpu.VMEM((NCHUNK_W, CHUNK), jnp.int32),
            pltpu.VMEM((2, CHUNK, EMBED_DIM), jnp.float32),
            pltpu.VMEM((BPW, EMBED_DIM), jnp.float32),
            pltpu.SemaphoreType.DMA,
            pltpu.SemaphoreType.DMA,
        ],
    )
    def kb(r_hbm, idx_hbm, out_hbm, idx_v, rows_v, pooled_v, sem0, sem1):
        wid = lax.axis_index("s") * NC + lax.axis_index("c")
        cbase = wid * NCHUNK_W
        pltpu.sync_copy(idx_hbm.at[pl.ds(cbase, NCHUNK_W)], idx_v)

        def start(c, slot, sem):
            pltpu.async_copy(r_hbm.at[idx_v.at[c]], rows_v.at[slot], sem)

        def wait(c, slot, sem):
            pltpu.make_async_copy(
                r_hbm.at[idx_v.at[c]], rows_v.at[slot], sem
            ).wait()

        def reduce_chunk(c, slot, accs):
            s_raw = (SEQ - (CHUNK * c) % SEQ) % SEQ
            s = jnp.minimum(s_raw, CHUNK)
            flush = jnp.logical_and(s_raw < CHUNK,
                                    jnp.logical_or(c > 0, s_raw > 0))

            def red(rr, a):
                return tuple(
                    a[j] + rows_v[slot, rr, pl.ds(16 * j, 16)]
                    for j in range(NV)
                )

            accs = lax.fori_loop(jnp.int32(0), s, red, accs)

            @pl.when(flush)
            def _():
                rf = (CHUNK * c + s - 1) // SEQ
                for j in range(NV):
                    pooled_v[rf, pl.ds(16 * j, 16)] = accs[j] * INV_SEQ

            accs = tuple(
                jnp.where(flush, jnp.zeros((LANES,), jnp.float32), a)
                for a in accs
            )
            return lax.fori_loop(s, jnp.int32(CHUNK), red, accs)

        start(0, 0, sem0)

        def body(cp, accs):
            c0 = 2 * cp
            start(c0 + 1, 1, sem1)
            wait(c0, 0, sem0)
            accs = reduce_chunk(c0, 0, accs)

            @pl.when(c0 + 2 < NCHUNK_W)
            def _():
                start(c0 + 2, 0, sem0)

            wait(c0 + 1, 1, sem1)
            return reduce_chunk(c0 + 1, 1, accs)

        accs = lax.fori_loop(
            0, NCHUNK_W // 2, body,
            tuple(jnp.zeros((LANES,), jnp.float32) for _ in range(NV)),
        )
        for j in range(NV):
            pooled_v[BPW - 1, pl.ds(16 * j, 16)] = accs[j] * INV_SEQ
        pltpu.sync_copy(pooled_v, out_hbm.at[pl.ds(wid * BPW, BPW)])

    return kb(r64, gidx)


def _classifier_tc(pooled, W, b2):
    B, D = pooled.shape
    C = W.shape[0]
    BM = 512

    def mm(x_ref, w_ref, b_ref, o_ref):
        o_ref[...] = (
            lax.dot_general(
                x_ref[...],
                w_ref[...],
                (((1,), (1,)), ((), ())),
                preferred_element_type=jnp.float32,
            )
            + b_ref[...]
        )

    return pl.pallas_call(
        mm,
        grid=(B // BM,),
        in_specs=[
            pl.BlockSpec((BM, D), lambda i: (i, 0)),
            pl.BlockSpec((C, D), lambda i: (0, 0)),
            pl.BlockSpec((1, C), lambda i: (0, 0)),
        ],
        out_specs=pl.BlockSpec((BM, C), lambda i: (i, 0)),
        out_shape=jax.ShapeDtypeStruct((B, C), jnp.float32),
    )(pooled, W, b2)


def kernel(x_data, table, W, b):
    x = x_data.astype(jnp.int32)
    bb = x >> 11
    rr = x & (BR - 1)
    gidx = ((bb >> 1) << 12) + 2 * rr + (bb & 1)
    gidx = gidx.reshape(BATCH * SEQ // CHUNK, CHUNK)
    table_t = jnp.swapaxes(table, 0, 1)
    r = _repack_tc(table_t)
    r64 = r.reshape(2 * RROWS, EMBED_DIM)
    pooled = _pool_sc(r64, gidx)
    return _classifier_tc(pooled, W, b.reshape(1, -1))
